# baseline (speedup 1.0000x reference)
import functools

import jax
import jax.numpy as jnp
from jax import lax
from jax.experimental import pallas as pl
from jax.experimental.pallas import tpu as pltpu
from jax.experimental.pallas import tpu_sc as plsc

B, S, D, H, NV, DR = 1, 2048, 1024, 16, 8192, 64
DH = D // H
SB = 256
NSB = S // SB
NEG = -1e30


def _proj_body(x_ref, wq_ref, bq_ref, wk_ref, bk_ref, wvq_ref, bvq_ref,
               vkt_ref, q_ref, k_ref, idx_ref):
    x = x_ref[...]
    q_ref[...] = jnp.dot(x, wq_ref[...],
                         preferred_element_type=jnp.float32) + bq_ref[...]
    k_ref[...] = jnp.dot(x, wk_ref[...],
                         preferred_element_type=jnp.float32) + bk_ref[...]
    vq = jnp.dot(x, wvq_ref[...],
                 preferred_element_type=jnp.float32) + bvq_ref[...]
    sim = jnp.dot(vq, vkt_ref[...], preferred_element_type=jnp.float32)
    m = jnp.max(sim, axis=1, keepdims=True)
    col = lax.broadcasted_iota(jnp.int32, sim.shape, 1)
    cand = jnp.where(sim == m, col, NV)
    idx_ref[0, 0, :] = jnp.min(cand, axis=1)


def _projections(x, Wq, bq, Wk, bk, Wvq, bvq, v_keys_t):
    full = lambda shape: pl.BlockSpec(shape, lambda i: (0,) * len(shape))
    return pl.pallas_call(
        _proj_body,
        grid=(NSB,),
        in_specs=[
            pl.BlockSpec((SB, D), lambda i: (i, 0)),
            full((D, D)), full((1, D)),
            full((D, D)), full((1, D)),
            full((D, DR)), full((1, DR)),
            full((DR, NV)),
        ],
        out_specs=[
            pl.BlockSpec((SB, D), lambda i: (i, 0)),
            pl.BlockSpec((SB, D), lambda i: (i, 0)),
            pl.BlockSpec((1, 1, SB), lambda i: (i, 0, 0)),
        ],
        out_shape=[
            jax.ShapeDtypeStruct((S, D), jnp.float32),
            jax.ShapeDtypeStruct((S, D), jnp.float32),
            jax.ShapeDtypeStruct((NSB, 1, SB), jnp.int32),
        ],
    )(x, Wq, bq.reshape(1, D), Wk, bk.reshape(1, D),
      Wvq, bvq.reshape(1, DR), v_keys_t)


_NW = 32
_BPW = S // _NW


def _sc_gather(idx, table):
    mesh = plsc.VectorSubcoreMesh(core_axis_name="c", subcore_axis_name="s")

    @functools.partial(
        pl.kernel, mesh=mesh,
        out_type=jax.ShapeDtypeStruct((S, D), jnp.float32),
        scratch_types=[
            pltpu.VMEM((_BPW,), jnp.int32),
            pltpu.VMEM((_BPW, D), jnp.float32),
            pltpu.SemaphoreType.DMA,
        ],
    )
    def k(idx_hbm, table_hbm, out_hbm, idx_v, rows_v, sem):
        wid = lax.axis_index("s") * 2 + lax.axis_index("c")
        base = wid * _BPW
        pltpu.sync_copy(idx_hbm.at[pl.ds(base, _BPW)], idx_v)
        pltpu.async_copy(table_hbm.at[idx_v], rows_v, sem).wait()
        pltpu.sync_copy(rows_v, out_hbm.at[pl.ds(base, _BPW)])

    return k(idx, table)


RB = 512
NRB = S // RB


def _attn_body(q_ref, k_ref, x_ref, vs_ref, o_ref):
    r = pl.program_id(1)
    d = (lax.broadcasted_iota(jnp.int32, (RB, RB), 0)
         - lax.broadcasted_iota(jnp.int32, (RB, RB), 1))
    q0 = q_ref[:, :DH] * 0.125
    q1 = q_ref[:, DH:] * 0.125

    def one(q, kh, vh, valid, m, l, a):
        s = lax.dot_general(q, kh, (((1,), (1,)), ((), ())),
                            preferred_element_type=jnp.float32)
        s = jnp.where(valid, s, NEG)
        mn = jnp.maximum(m, jnp.max(s, axis=1, keepdims=True))
        p = jnp.exp(s - mn)
        alpha = jnp.exp(m - mn)
        l = l * alpha + jnp.sum(p, axis=1, keepdims=True)
        a = a * alpha + jnp.dot(p, vh, preferred_element_type=jnp.float32)
        return mn, l, a

    def body(cb, carry):
        m0, l0, a0, m1, l1, a1 = carry
        off = pl.ds(cb * RB, RB)
        kb = k_ref[off, :]
        vb = x_ref[off, :] * vs_ref[off, :]
        valid = d + (r - cb) * RB >= 0
        m0, l0, a0 = one(q0, kb[:, :DH], vb[:, :DH], valid, m0, l0, a0)
        m1, l1, a1 = one(q1, kb[:, DH:], vb[:, DH:], valid, m1, l1, a1)
        return m0, l0, a0, m1, l1, a1

    mi = jnp.full((RB, 1), NEG, jnp.float32)
    li = jnp.zeros((RB, 1), jnp.float32)
    ai = jnp.zeros((RB, DH), jnp.float32)
    m0, l0, a0, m1, l1, a1 = lax.fori_loop(
        0, r + 1, body, (mi, li, ai, mi, li, ai))
    o_ref[:, :DH] = a0 / l0
    o_ref[:, DH:] = a1 / l1


def _attention(q, k, x, v_sel):
    rows = pl.BlockSpec((RB, 2 * DH), lambda h, r: (r, h))
    full = pl.BlockSpec((S, 2 * DH), lambda h, r: (0, h))
    return pl.pallas_call(
        _attn_body,
        grid=(H // 2, NRB),
        in_specs=[rows, full, full, full],
        out_specs=rows,
        out_shape=jax.ShapeDtypeStruct((S, D), jnp.float32),
    )(q, k, x, v_sel)


def _out_body(a_ref, wo_ref, bo_ref, o_ref):
    o_ref[...] = jnp.dot(a_ref[...], wo_ref[...],
                         preferred_element_type=jnp.float32) + bo_ref[...]


def _out_proj(attn, Wo, bo):
    return pl.pallas_call(
        _out_body,
        out_shape=jax.ShapeDtypeStruct((S, D), jnp.float32),
    )(attn, Wo, bo.reshape(1, D))


def kernel(hidden_states, Wq, bq, Wk, bk, Wvq, bvq, v_keys, v_embed, Wo, bo):
    x = hidden_states.reshape(S, D)
    q, k, idx3 = _projections(x, Wq, bq, Wk, bk, Wvq, bvq, v_keys.T)
    v_sel = _sc_gather(idx3.reshape(S), v_embed)
    attn = _attention(q, k, x, v_sel)
    out = _out_proj(attn, Wo, bo)
    return out.reshape(B, S, D)


# speedup vs baseline: 1.3821x; 1.3821x over previous
import functools

import jax
import jax.numpy as jnp
from jax import lax
from jax.experimental import pallas as pl
from jax.experimental.pallas import tpu as pltpu
from jax.experimental.pallas import tpu_sc as plsc

B, S, D, H, NV, DR = 1, 2048, 1024, 16, 8192, 64
DH = D // H
SB = 256
NSB = S // SB
NEG = -1e30


def _proj_body(x_ref, wq_ref, bq_ref, wk_ref, bk_ref, wvq_ref, bvq_ref,
               vkt_ref, q_ref, k_ref, idx_ref):
    x = x_ref[...]
    q_ref[...] = jnp.dot(x, wq_ref[...],
                         preferred_element_type=jnp.float32) + bq_ref[...]
    k_ref[...] = jnp.dot(x, wk_ref[...],
                         preferred_element_type=jnp.float32) + bk_ref[...]
    vq = jnp.dot(x, wvq_ref[...],
                 preferred_element_type=jnp.float32) + bvq_ref[...]
    sim = jnp.dot(vq, vkt_ref[...], preferred_element_type=jnp.float32)
    m = jnp.max(sim, axis=1, keepdims=True)
    col = lax.broadcasted_iota(jnp.int32, sim.shape, 1)
    cand = jnp.where(sim == m, col, NV)
    idx_ref[0, 0, :] = jnp.min(cand, axis=1)


def _projections(x, Wq, bq, Wk, bk, Wvq, bvq, v_keys_t):
    full = lambda shape: pl.BlockSpec(shape, lambda i: (0,) * len(shape))
    return pl.pallas_call(
        _proj_body,
        grid=(NSB,),
        in_specs=[
            pl.BlockSpec((SB, D), lambda i: (i, 0)),
            full((D, D)), full((1, D)),
            full((D, D)), full((1, D)),
            full((D, DR)), full((1, DR)),
            full((DR, NV)),
        ],
        out_specs=[
            pl.BlockSpec((SB, D), lambda i: (i, 0)),
            pl.BlockSpec((SB, D), lambda i: (i, 0)),
            pl.BlockSpec((1, 1, SB), lambda i: (i, 0, 0)),
        ],
        out_shape=[
            jax.ShapeDtypeStruct((S, D), jnp.float32),
            jax.ShapeDtypeStruct((S, D), jnp.float32),
            jax.ShapeDtypeStruct((NSB, 1, SB), jnp.int32),
        ],
    )(x, Wq, bq.reshape(1, D), Wk, bk.reshape(1, D),
      Wvq, bvq.reshape(1, DR), v_keys_t)


_NW = 32
_BPW = S // _NW


def _sc_gather(idx, table):
    mesh = plsc.VectorSubcoreMesh(core_axis_name="c", subcore_axis_name="s")

    @functools.partial(
        pl.kernel, mesh=mesh,
        out_type=jax.ShapeDtypeStruct((S, D), jnp.float32),
        scratch_types=[
            pltpu.VMEM((_BPW,), jnp.int32),
            pltpu.VMEM((_BPW, D), jnp.float32),
            pltpu.SemaphoreType.DMA,
        ],
    )
    def k(idx_hbm, table_hbm, out_hbm, idx_v, rows_v, sem):
        wid = lax.axis_index("s") * 2 + lax.axis_index("c")
        base = wid * _BPW
        pltpu.sync_copy(idx_hbm.at[pl.ds(base, _BPW)], idx_v)
        pltpu.async_copy(table_hbm.at[idx_v], rows_v, sem).wait()
        pltpu.sync_copy(rows_v, out_hbm.at[pl.ds(base, _BPW)])

    return k(idx, table)


RB = 512
NRB = S // RB


def _attn_body(q_ref, k_ref, x_ref, vs_ref, o_ref):
    for r in range(NRB):
        w = (r + 1) * RB
        rows = slice(r * RB, (r + 1) * RB)
        valid = (lax.broadcasted_iota(jnp.int32, (RB, w), 0) + r * RB
                 >= lax.broadcasted_iota(jnp.int32, (RB, w), 1))
        for j in range(2):
            sl = slice(j * DH, (j + 1) * DH)
            q = q_ref[rows, sl] * 0.125
            s = lax.dot_general(q, k_ref[:w, sl], (((1,), (1,)), ((), ())),
                                preferred_element_type=jnp.float32)
            s = jnp.where(valid, s, NEG)
            m = jnp.max(s, axis=1, keepdims=True)
            p = jnp.exp(s - m)
            p = p / jnp.sum(p, axis=1, keepdims=True)
            v = x_ref[:w, sl] * vs_ref[:w, sl]
            o_ref[rows, sl] = jnp.dot(p, v, preferred_element_type=jnp.float32)


def _attention(q, k, x, v_sel):
    pair = pl.BlockSpec((S, 2 * DH), lambda h: (0, h))
    return pl.pallas_call(
        _attn_body,
        grid=(H // 2,),
        in_specs=[pair, pair, pair, pair],
        out_specs=pair,
        out_shape=jax.ShapeDtypeStruct((S, D), jnp.float32),
    )(q, k, x, v_sel)


def _out_body(a_ref, wo_ref, bo_ref, o_ref):
    o_ref[...] = jnp.dot(a_ref[...], wo_ref[...],
                         preferred_element_type=jnp.float32) + bo_ref[...]


def _out_proj(attn, Wo, bo):
    return pl.pallas_call(
        _out_body,
        out_shape=jax.ShapeDtypeStruct((S, D), jnp.float32),
    )(attn, Wo, bo.reshape(1, D))


def kernel(hidden_states, Wq, bq, Wk, bk, Wvq, bvq, v_keys, v_embed, Wo, bo):
    x = hidden_states.reshape(S, D)
    q, k, idx3 = _projections(x, Wq, bq, Wk, bk, Wvq, bvq, v_keys.T)
    v_sel = _sc_gather(idx3.reshape(S), v_embed)
    attn = _attention(q, k, x, v_sel)
    out = _out_proj(attn, Wo, bo)
    return out.reshape(B, S, D)
